# baseline (speedup 1.0000x reference)
import functools

import jax
import jax.numpy as jnp
from jax import lax
from jax.experimental import pallas as pl
from jax.experimental.pallas import tpu as pltpu
from jax.experimental.pallas import tpu_sc as plsc

N_NODES = 50000
N_EDGES = 800000
LATENT = 64
STEPS = 3
R_MAX = 5.0
N_BASES = 8

NBK = 256
EBK = 1024
N_PAD = 50176
E_PAD = 819200
CHUNK = 128
N_CHUNKS = E_PAD // CHUNK
NC, NS = 2, 16
NW = NC * NS
CPW = N_CHUNKS // NW
CPT = N_CHUNKS // NS
NROWS_T = N_PAD // NS
IBLK = 40

def _get_mesh():
    return plsc.VectorSubcoreMesh(
        core_axis_name="c", subcore_axis_name="s",
        num_cores=NC, num_subcores=NS)


def _silu(x):
    return x * jax.nn.sigmoid(x)


def _dot(a, b, preferred_element_type=jnp.float32):
    return jnp.dot(a, b, preferred_element_type=preferred_element_type,
                   precision=lax.Precision.HIGHEST)


@functools.cache
def _make_gather2(d):
    @functools.partial(
        pl.kernel,
        out_type=jax.ShapeDtypeStruct((E_PAD, 2 * d), jnp.float32),
        mesh=_get_mesh(),
        scratch_types=[
            pltpu.VMEM((CPW, CHUNK), jnp.int32),
            pltpu.VMEM((CPW, CHUNK), jnp.int32),
            pltpu.VMEM((2, CHUNK, d), jnp.float32),
            pltpu.VMEM((2, CHUNK, d), jnp.float32),
            [pltpu.SemaphoreType.DMA] * 2,
            [pltpu.SemaphoreType.DMA] * 2,
        ],
        compiler_params=pltpu.CompilerParams(use_tc_tiling_on_sc=False),
    )
    def gather2(ta, tb, sidx, ridx, outab, idx_s, idx_r, bufa, bufb,
                gsem, wsem):
        wid = lax.axis_index("s") * NC + lax.axis_index("c")
        cbase = wid * CPW
        pltpu.sync_copy(sidx.at[pl.ds(cbase, CPW)], idx_s)
        pltpu.sync_copy(ridx.at[pl.ds(cbase, CPW)], idx_r)

        def fire(j, slot):
            pltpu.async_copy(ta.at[idx_s.at[j]], bufa.at[slot], gsem[slot])
            pltpu.async_copy(tb.at[idx_r.at[j]], bufb.at[slot], gsem[slot])

        def out_refs(j):
            ebase = (cbase + j) * CHUNK
            return (outab.at[pl.ds(ebase, CHUNK), pl.ds(0, d)],
                    outab.at[pl.ds(ebase, CHUNK), pl.ds(d, d)])

        def wait_gathers(j, slot):
            pltpu.make_async_copy(
                ta.at[idx_s.at[j]], bufa.at[slot], gsem[slot]).wait()
            pltpu.make_async_copy(
                tb.at[idx_r.at[j]], bufb.at[slot], gsem[slot]).wait()

        def drain_writes(j, slot):
            oa, ob = out_refs(j)
            pltpu.make_async_copy(bufa.at[slot], oa, wsem[slot]).wait()
            pltpu.make_async_copy(bufb.at[slot], ob, wsem[slot]).wait()

        def issue_writes(j, slot):
            oa, ob = out_refs(j)
            pltpu.async_copy(bufa.at[slot], oa, wsem[slot])
            pltpu.async_copy(bufb.at[slot], ob, wsem[slot])

        fire(0, 0)

        @pl.loop(0, CPW // 2)
        def _(jj):
            j0 = jj * 2
            wait_gathers(j0, 0)

            @pl.when(jj > 0)
            def _():
                drain_writes(j0 - 1, 1)

            fire(j0 + 1, 1)
            issue_writes(j0, 0)

            wait_gathers(j0 + 1, 1)
            drain_writes(j0, 0)

            @pl.when(jj < CPW // 2 - 1)
            def _():
                fire(j0 + 2, 0)

            issue_writes(j0 + 1, 1)

        drain_writes(CPW - 1, 1)

    return gather2


def _gather2_64(ta, tb, sidx, ridx):
    return _make_gather2(64)(ta, tb, sidx, ridx)


@functools.cache
def _make_scatter():
    @functools.partial(
        pl.kernel,
        out_type=(jax.ShapeDtypeStruct((N_PAD, 32), jnp.float32),
                  jax.ShapeDtypeStruct((N_PAD, 32), jnp.float32)),
        mesh=_get_mesh(),
        scratch_types=[
            pltpu.VMEM_SHARED((N_PAD, 32), jnp.float32),
            pltpu.VMEM((IBLK, CHUNK), jnp.int32),
            pltpu.VMEM((2, CHUNK, 32), jnp.float32),
            [pltpu.SemaphoreType.DMA] * 2,
        ],
        compiler_params=pltpu.CompilerParams(use_tc_tiling_on_sc=False),
    )
    def scatter_agg(msga, msgb, ridx, zinit, agga, aggb, spmem, idx_v, buf,
                    sem):
        c = lax.axis_index("c")
        s = lax.axis_index("s")
        rbase = s * NROWS_T

        pltpu.sync_copy(zinit, spmem.at[pl.ds(rbase, NROWS_T)])
        plsc.subcore_barrier()

        def accum(msg):
            def fire(cb, j, slot):
                pltpu.async_copy(
                    msg.at[pl.ds((cb + j) * CHUNK, CHUNK)],
                    buf.at[slot], sem[slot])

            def wait(cb, j, slot):
                pltpu.make_async_copy(
                    msg.at[pl.ds((cb + j) * CHUNK, CHUNK)],
                    buf.at[slot], sem[slot]).wait()

            @pl.loop(0, CPT // IBLK)
            def _(b):
                cb = s * CPT + b * IBLK
                pltpu.sync_copy(ridx.at[pl.ds(cb, IBLK)], idx_v)
                fire(cb, 0, 0)

                @pl.loop(0, IBLK // 2)
                def _(jj):
                    j0 = jj * 2
                    wait(cb, j0, 0)
                    fire(cb, j0 + 1, 1)
                    pltpu.sync_copy(
                        buf.at[0], spmem.at[idx_v.at[j0]], add=True)
                    wait(cb, j0 + 1, 1)

                    @pl.when(jj < IBLK // 2 - 1)
                    def _():
                        fire(cb, j0 + 2, 0)

                    pltpu.sync_copy(
                        buf.at[1], spmem.at[idx_v.at[j0 + 1]], add=True)

        @pl.when(c == 0)
        def _():
            accum(msga)

        @pl.when(c == 1)
        def _():
            accum(msgb)

        plsc.subcore_barrier()

        @pl.when(c == 0)
        def _():
            pltpu.sync_copy(spmem.at[pl.ds(rbase, NROWS_T)],
                            agga.at[pl.ds(rbase, NROWS_T)])

        @pl.when(c == 1)
        def _():
            pltpu.sync_copy(spmem.at[pl.ds(rbase, NROWS_T)],
                            aggb.at[pl.ds(rbase, NROWS_T)])

    return scatter_agg


def _scatter_agg(msga, msgb, ridx, zinit):
    return _make_scatter()(msga, msgb, ridx, zinit)


def _node_encode_body(codes, ctab, bias, w1a, w1b, h_out, a_out, b_out):
    iot = lax.broadcasted_iota(jnp.int32, (NBK, 64), 1)
    m = jnp.zeros((NBK, 64), jnp.float32)
    for k in range(4):
        m = m + (codes[:, k:k + 1] == iot).astype(jnp.float32)
    h = _silu(_dot(m, ctab[...], preferred_element_type=jnp.float32)
              + bias[...])
    h_out[...] = h
    a_out[...] = _dot(h, w1a[...], preferred_element_type=jnp.float32)
    b_out[...] = _dot(h, w1b[...], preferred_element_type=jnp.float32)


def _edge_encode_body(pp, bondf, wr, wsl, wsq, b2tab, bias, sel,
                      e_out, fc_out):
    d = pp[:, 0:64] - pp[:, 64:128]
    r2 = jnp.sum(d[:, 0:16] * d[:, 0:16], axis=1, keepdims=True)
    r = jnp.sqrt(r2)
    rs = jnp.maximum(r, 1e-6)
    u = d[:, 0:16] / rs
    n = (lax.broadcasted_iota(jnp.int32, (EBK, N_BASES), 1) + 1
         ).astype(jnp.float32)
    rbf = jnp.sqrt(2.0 / R_MAX) * jnp.sin(n * (jnp.pi / R_MAX) * rs) / rs
    acc = _dot(rbf, wr[...], preferred_element_type=jnp.float32)
    del sel
    xb = jnp.broadcast_to(u[:, 0:1], (EBK, 64))
    yb = jnp.broadcast_to(u[:, 1:2], (EBK, 64))
    zb = jnp.broadcast_to(u[:, 2:3], (EBK, 64))
    acc = acc + xb * wsl[0:1, :] + yb * wsl[1:2, :] + zb * wsl[2:3, :]
    acc = acc + (xb * yb) * wsq[0:1, :] + (yb * zb) * wsq[1:2, :] \
        + (zb * zb) * wsq[2:3, :] + (xb * zb) * wsq[3:4, :] \
        + (xb * xb) * wsq[4:5, :] + (yb * yb) * wsq[5:6, :]
    acc = acc + b2tab[0:1, :] + bondf[...] * (b2tab[1:2, :] - b2tab[0:1, :])
    e_out[...] = _silu(acc + bias[...])
    fc = 0.5 * (jnp.cos((jnp.pi / R_MAX) * jnp.minimum(r, R_MAX)) + 1.0)
    fc = fc * (r < R_MAX).astype(jnp.float32)
    gid = (pl.program_id(0) * EBK
           + lax.broadcasted_iota(jnp.int32, (EBK, 1), 0))
    fc_out[...] = fc * (gid < N_EDGES).astype(jnp.float32)


def _edge_step_body(xab, e, fc, w1c, b1, w2m, b2c, w2g,
                    e_out, ma_out, mb_out):
    pre = (xab[:, 0:64] + xab[:, 64:128]
           + _dot(e[...], w1c[...], preferred_element_type=jnp.float32)
           + b1[...])
    m = _silu(pre)
    e_out[...] = e[...] + m
    m2 = _dot(m, w2m[...], preferred_element_type=jnp.float32) \
        + b2c[:, 0:64]
    g = _dot(m, w2g[...], preferred_element_type=jnp.float32) \
        + b2c[:, 64:65]
    msg = m2 * jax.nn.sigmoid(g) * fc[...]
    ma_out[...] = msg[:, 0:32]
    mb_out[...] = msg[:, 32:64]


def _node_step_body(h, agga, aggb, wn1h, wn1a, wn1b, bn1, wn2, bn2,
                    w1a, w1b, h_out, a_out, b_out):
    t = (_dot(h[...], wn1h[...], preferred_element_type=jnp.float32)
         + _dot(agga[...], wn1a[...], preferred_element_type=jnp.float32)
         + _dot(aggb[...], wn1b[...], preferred_element_type=jnp.float32)
         + bn1[...])
    hn = h[...] + _dot(_silu(t), wn2[...],
                          preferred_element_type=jnp.float32) + bn2[...]
    h_out[...] = hn
    a_out[...] = _dot(hn, w1a[...], preferred_element_type=jnp.float32)
    b_out[...] = _dot(hn, w1b[...], preferred_element_type=jnp.float32)


def _node_last_body(h, agga, aggb, wn1h, wn1a, wn1b, bn1, wn2, bn2,
                    wh1, bh1, wh2, bh2, out):
    t = (_dot(h[...], wn1h[...], preferred_element_type=jnp.float32)
         + _dot(agga[...], wn1a[...], preferred_element_type=jnp.float32)
         + _dot(aggb[...], wn1b[...], preferred_element_type=jnp.float32)
         + bn1[...])
    hn = h[...] + _dot(_silu(t), wn2[...],
                          preferred_element_type=jnp.float32) + bn2[...]
    y = _silu(_dot(hn, wh1[...], preferred_element_type=jnp.float32)
              + bh1[...])
    out[...] = _dot(y, wh2[...], preferred_element_type=jnp.float32) \
        + bh2[...]


def _row_spec(rows, cols):
    return pl.BlockSpec((rows, cols), lambda i: (i, 0))


def _rep_spec(shape):
    return pl.BlockSpec(shape, lambda i: tuple(0 for _ in shape))


def kernel(pos, atom_type_index, atom_code_index, residue_code_index,
           residue_sequence_index, bond_mask, senders, receivers, batch,
           num_graphs, c_noise, c_in, params):
    del batch, num_graphs, c_noise
    f32 = jnp.float32

    up64 = jnp.zeros((N_PAD, 64), f32)
    up64 = up64.at[:N_NODES, :3].set(pos / c_in[0])

    codes = jnp.stack([
        atom_type_index,
        20 + atom_code_index,
        30 + residue_code_index,
        jnp.full((N_NODES,), 55, jnp.int32),
    ], axis=1)
    codes = jnp.concatenate(
        [codes, jnp.zeros((N_PAD - N_NODES, 4), jnp.int32)], axis=0)

    epad = E_PAD - N_EDGES
    s2d = jnp.concatenate(
        [senders, jnp.zeros((epad,), jnp.int32)]).reshape(N_CHUNKS, CHUNK)
    r2d = jnp.concatenate(
        [receivers, jnp.zeros((epad,), jnp.int32)]).reshape(N_CHUNKS, CHUNK)
    bondf = jnp.concatenate(
        [bond_mask.astype(f32), jnp.zeros((epad,), f32)]).reshape(E_PAD, 1)

    p = params
    w_node = p['W_node_enc']
    ctab = jnp.zeros((64, 64), f32)
    ctab = ctab.at[0:20].set(p['emb_atom_type'] @ w_node[0:32])
    ctab = ctab.at[20:30].set(p['emb_atom_code'] @ w_node[32:48])
    ctab = ctab.at[30:55].set(p['emb_res_code'] @ w_node[48:64])
    ctab = ctab.at[55:56].set(p['emb_res_idx'] @ w_node[64:80])
    b_node = p['b_node_enc'].reshape(1, 64)

    w_edge = p['W_edge_enc']
    wr = w_edge[0:8]
    ws = w_edge[8:17]
    s3, s15, s5 = jnp.sqrt(3.0), jnp.sqrt(15.0), jnp.sqrt(5.0)
    wsl = jnp.stack([s3 * ws[1], s3 * ws[2], s3 * ws[3]])
    wsq = jnp.stack([s15 * ws[4], s15 * ws[5], 1.5 * s5 * ws[6],
                     s15 * ws[7], 0.5 * s15 * ws[8], -0.5 * s15 * ws[8]])
    b2tab = p['emb_bond'] @ w_edge[17:33]
    b_edge = (p['b_edge_enc'] + ws[0] - 0.5 * s5 * ws[6]).reshape(1, 64)
    sel = jnp.zeros((16, 192), f32)
    sel = sel.at[0, 0:64].set(1.0).at[1, 64:128].set(1.0)
    sel = sel.at[2, 128:192].set(1.0)

    steps_w = []
    for s in range(STEPS):
        ps_ = p['steps'][s]
        steps_w.append(dict(
            w1a=ps_['W_e1'][0:64], w1b=ps_['W_e1'][64:128],
            w1c=ps_['W_e1'][128:192], b1=ps_['b_e1'].reshape(1, 64),
            w2m=ps_['W_e2'][:, 0:64], w2g=ps_['W_e2'][:, 64:65],
            b2c=jnp.zeros((1, 128), f32).at[0, :65].set(ps_['b_e2']),
            wn1h=ps_['W_n1'][0:64], wn1a=ps_['W_n1'][64:96],
            wn1b=ps_['W_n1'][96:128], bn1=ps_['b_n1'].reshape(1, 64),
            wn2=ps_['W_n2'], bn2=ps_['b_n2'].reshape(1, 64),
        ))
    wh2 = jnp.zeros((64, 8), f32).at[:, :3].set(p['W_h2'])
    bh2 = jnp.zeros((1, 8), f32).at[0, :3].set(p['b_h2'])
    bh1 = p['b_h1'].reshape(1, 64)

    zinit = jnp.zeros((NROWS_T, 32), f32)

    ngrid = N_PAD // NBK
    egrid = E_PAD // EBK
    nfull = _row_spec(NBK, 64)
    nhalf = _row_spec(NBK, 32)
    efull = _row_spec(EBK, 64)
    ehalf = _row_spec(EBK, 32)
    e1col = _row_spec(EBK, 1)
    w64 = _rep_spec((64, 64))
    b64 = _rep_spec((1, 64))

    h, a_mat, b_mat = pl.pallas_call(
        _node_encode_body,
        grid=(ngrid,),
        in_specs=[_row_spec(NBK, 4), w64, b64, w64, w64],
        out_specs=[nfull, nfull, nfull],
        out_shape=[jax.ShapeDtypeStruct((N_PAD, 64), f32)] * 3,
    )(codes, ctab, b_node, steps_w[0]['w1a'], steps_w[0]['w1b'])

    pp = _gather2_64(up64, up64, s2d, r2d)
    e, fc = pl.pallas_call(
        _edge_encode_body,
        grid=(egrid,),
        in_specs=[_row_spec(EBK, 128), e1col,
                  _rep_spec((8, 64)), _rep_spec((3, 64)), _rep_spec((6, 64)),
                  _rep_spec((2, 64)), b64, _rep_spec((16, 192))],
        out_specs=[efull, e1col],
        out_shape=[jax.ShapeDtypeStruct((E_PAD, 64), f32),
                   jax.ShapeDtypeStruct((E_PAD, 1), f32)],
    )(pp, bondf, wr, wsl, wsq, b2tab, b_edge, sel)

    out = None
    for s in range(STEPS):
        sw = steps_w[s]
        xab = _gather2_64(a_mat, b_mat, s2d, r2d)
        e, msga, msgb = pl.pallas_call(
            _edge_step_body,
            grid=(egrid,),
            in_specs=[_row_spec(EBK, 128), efull, e1col, w64, b64, w64,
                      _rep_spec((1, 128)), _rep_spec((64, 1))],
            out_specs=[efull, ehalf, ehalf],
            out_shape=[jax.ShapeDtypeStruct((E_PAD, 64), f32),
                       jax.ShapeDtypeStruct((E_PAD, 32), f32),
                       jax.ShapeDtypeStruct((E_PAD, 32), f32)],
            input_output_aliases={1: 0},
        )(xab, e, fc, sw['w1c'], sw['b1'], sw['w2m'], sw['b2c'], sw['w2g'])

        agga, aggb = _scatter_agg(msga, msgb, r2d, zinit)

        if s < STEPS - 1:
            nw = steps_w[s + 1]
            h, a_mat, b_mat = pl.pallas_call(
                _node_step_body,
                grid=(ngrid,),
                in_specs=[nfull, nhalf, nhalf, w64, _rep_spec((32, 64)),
                          _rep_spec((32, 64)), b64, w64, b64, w64, w64],
                out_specs=[nfull, nfull, nfull],
                out_shape=[jax.ShapeDtypeStruct((N_PAD, 64), f32)] * 3,
            )(h, agga, aggb, sw['wn1h'], sw['wn1a'], sw['wn1b'], sw['bn1'],
              sw['wn2'], sw['bn2'], nw['w1a'], nw['w1b'])
        else:
            out = pl.pallas_call(
                _node_last_body,
                grid=(ngrid,),
                in_specs=[nfull, nhalf, nhalf, w64, _rep_spec((32, 64)),
                          _rep_spec((32, 64)), b64, w64, b64, w64, b64,
                          _rep_spec((64, 8)), _rep_spec((1, 8))],
                out_specs=[_row_spec(NBK, 8)],
                out_shape=[jax.ShapeDtypeStruct((N_PAD, 8), f32)],
            )(h, agga, aggb, sw['wn1h'], sw['wn1a'], sw['wn1b'], sw['bn1'],
              sw['wn2'], sw['bn2'], p['W_h1'], bh1, wh2, bh2)[0]

    return out[:N_NODES, :3]


# speedup vs baseline: 1.1300x; 1.1300x over previous
import functools

import jax
import jax.numpy as jnp
from jax import lax
from jax.experimental import pallas as pl
from jax.experimental.pallas import tpu as pltpu
from jax.experimental.pallas import tpu_sc as plsc

N_NODES = 50000
N_EDGES = 800000
LATENT = 64
STEPS = 3
R_MAX = 5.0
N_BASES = 8

NBK = 256
EBK = 1024
N_PAD = 50176
E_PAD = 819200
CHUNK = 128
N_CHUNKS = E_PAD // CHUNK
NC, NS = 2, 16
NW = NC * NS
CPW = N_CHUNKS // NW
CPT = N_CHUNKS // NS
NROWS_T = N_PAD // NS
IBLK = 40

def _get_mesh():
    return plsc.VectorSubcoreMesh(
        core_axis_name="c", subcore_axis_name="s",
        num_cores=NC, num_subcores=NS)


def _silu(x):
    return x * jax.nn.sigmoid(x)


def _dot(a, b, preferred_element_type=jnp.float32):
    return jnp.dot(a, b, preferred_element_type=preferred_element_type)


@functools.cache
def _make_gather2(d):
    @functools.partial(
        pl.kernel,
        out_type=jax.ShapeDtypeStruct((E_PAD, 2 * d), jnp.float32),
        mesh=_get_mesh(),
        scratch_types=[
            pltpu.VMEM((CPW, CHUNK), jnp.int32),
            pltpu.VMEM((CPW, CHUNK), jnp.int32),
            pltpu.VMEM((2, CHUNK, d), jnp.float32),
            pltpu.VMEM((2, CHUNK, d), jnp.float32),
            [pltpu.SemaphoreType.DMA] * 2,
            [pltpu.SemaphoreType.DMA] * 2,
        ],
        compiler_params=pltpu.CompilerParams(use_tc_tiling_on_sc=False),
    )
    def gather2(ta, tb, sidx, ridx, outab, idx_s, idx_r, bufa, bufb,
                gsem, wsem):
        wid = lax.axis_index("s") * NC + lax.axis_index("c")
        cbase = wid * CPW
        pltpu.sync_copy(sidx.at[pl.ds(cbase, CPW)], idx_s)
        pltpu.sync_copy(ridx.at[pl.ds(cbase, CPW)], idx_r)

        def fire(j, slot):
            pltpu.async_copy(ta.at[idx_s.at[j]], bufa.at[slot], gsem[slot])
            pltpu.async_copy(tb.at[idx_r.at[j]], bufb.at[slot], gsem[slot])

        def out_refs(j):
            ebase = (cbase + j) * CHUNK
            return (outab.at[pl.ds(ebase, CHUNK), pl.ds(0, d)],
                    outab.at[pl.ds(ebase, CHUNK), pl.ds(d, d)])

        def wait_gathers(j, slot):
            pltpu.make_async_copy(
                ta.at[idx_s.at[j]], bufa.at[slot], gsem[slot]).wait()
            pltpu.make_async_copy(
                tb.at[idx_r.at[j]], bufb.at[slot], gsem[slot]).wait()

        def drain_writes(j, slot):
            oa, ob = out_refs(j)
            pltpu.make_async_copy(bufa.at[slot], oa, wsem[slot]).wait()
            pltpu.make_async_copy(bufb.at[slot], ob, wsem[slot]).wait()

        def issue_writes(j, slot):
            oa, ob = out_refs(j)
            pltpu.async_copy(bufa.at[slot], oa, wsem[slot])
            pltpu.async_copy(bufb.at[slot], ob, wsem[slot])

        fire(0, 0)

        @pl.loop(0, CPW // 2)
        def _(jj):
            j0 = jj * 2
            wait_gathers(j0, 0)

            @pl.when(jj > 0)
            def _():
                drain_writes(j0 - 1, 1)

            fire(j0 + 1, 1)
            issue_writes(j0, 0)

            wait_gathers(j0 + 1, 1)
            drain_writes(j0, 0)

            @pl.when(jj < CPW // 2 - 1)
            def _():
                fire(j0 + 2, 0)

            issue_writes(j0 + 1, 1)

        drain_writes(CPW - 1, 1)

    return gather2


def _gather2_64(ta, tb, sidx, ridx):
    return _make_gather2(64)(ta, tb, sidx, ridx)


@functools.cache
def _make_scatter():
    @functools.partial(
        pl.kernel,
        out_type=(jax.ShapeDtypeStruct((N_PAD, 32), jnp.float32),
                  jax.ShapeDtypeStruct((N_PAD, 32), jnp.float32)),
        mesh=_get_mesh(),
        scratch_types=[
            pltpu.VMEM_SHARED((N_PAD, 32), jnp.float32),
            pltpu.VMEM((IBLK, CHUNK), jnp.int32),
            pltpu.VMEM((2, CHUNK, 32), jnp.float32),
            [pltpu.SemaphoreType.DMA] * 2,
        ],
        compiler_params=pltpu.CompilerParams(use_tc_tiling_on_sc=False),
    )
    def scatter_agg(msga, msgb, ridx, zinit, agga, aggb, spmem, idx_v, buf,
                    sem):
        c = lax.axis_index("c")
        s = lax.axis_index("s")
        rbase = s * NROWS_T

        pltpu.sync_copy(zinit, spmem.at[pl.ds(rbase, NROWS_T)])
        plsc.subcore_barrier()

        def accum(msg):
            def fire(cb, j, slot):
                pltpu.async_copy(
                    msg.at[pl.ds((cb + j) * CHUNK, CHUNK)],
                    buf.at[slot], sem[slot])

            def wait(cb, j, slot):
                pltpu.make_async_copy(
                    msg.at[pl.ds((cb + j) * CHUNK, CHUNK)],
                    buf.at[slot], sem[slot]).wait()

            @pl.loop(0, CPT // IBLK)
            def _(b):
                cb = s * CPT + b * IBLK
                pltpu.sync_copy(ridx.at[pl.ds(cb, IBLK)], idx_v)
                fire(cb, 0, 0)

                @pl.loop(0, IBLK // 2)
                def _(jj):
                    j0 = jj * 2
                    wait(cb, j0, 0)
                    fire(cb, j0 + 1, 1)
                    pltpu.sync_copy(
                        buf.at[0], spmem.at[idx_v.at[j0]], add=True)
                    wait(cb, j0 + 1, 1)

                    @pl.when(jj < IBLK // 2 - 1)
                    def _():
                        fire(cb, j0 + 2, 0)

                    pltpu.sync_copy(
                        buf.at[1], spmem.at[idx_v.at[j0 + 1]], add=True)

        @pl.when(c == 0)
        def _():
            accum(msga)

        @pl.when(c == 1)
        def _():
            accum(msgb)

        plsc.subcore_barrier()

        @pl.when(c == 0)
        def _():
            pltpu.sync_copy(spmem.at[pl.ds(rbase, NROWS_T)],
                            agga.at[pl.ds(rbase, NROWS_T)])

        @pl.when(c == 1)
        def _():
            pltpu.sync_copy(spmem.at[pl.ds(rbase, NROWS_T)],
                            aggb.at[pl.ds(rbase, NROWS_T)])

    return scatter_agg


def _scatter_agg(msga, msgb, ridx, zinit):
    return _make_scatter()(msga, msgb, ridx, zinit)


def _node_encode_body(codes, ctab, bias, w1a, w1b, h_out, a_out, b_out):
    iot = lax.broadcasted_iota(jnp.int32, (NBK, 64), 1)
    m = jnp.zeros((NBK, 64), jnp.float32)
    for k in range(4):
        m = m + (codes[:, k:k + 1] == iot).astype(jnp.float32)
    h = _silu(_dot(m, ctab[...], preferred_element_type=jnp.float32)
              + bias[...])
    h_out[...] = h
    a_out[...] = _dot(h, w1a[...], preferred_element_type=jnp.float32)
    b_out[...] = _dot(h, w1b[...], preferred_element_type=jnp.float32)


def _edge_encode_body(pp, bondf, wr, wsl, wsq, b2tab, bias, sel,
                      e_out, fc_out):
    d = pp[:, 0:64] - pp[:, 64:128]
    r2 = jnp.sum(d[:, 0:16] * d[:, 0:16], axis=1, keepdims=True)
    r = jnp.sqrt(r2)
    rs = jnp.maximum(r, 1e-6)
    u = d[:, 0:16] / rs
    n = (lax.broadcasted_iota(jnp.int32, (EBK, N_BASES), 1) + 1
         ).astype(jnp.float32)
    rbf = jnp.sqrt(2.0 / R_MAX) * jnp.sin(n * (jnp.pi / R_MAX) * rs) / rs
    acc = _dot(rbf, wr[...], preferred_element_type=jnp.float32)
    del sel
    xb = jnp.broadcast_to(u[:, 0:1], (EBK, 64))
    yb = jnp.broadcast_to(u[:, 1:2], (EBK, 64))
    zb = jnp.broadcast_to(u[:, 2:3], (EBK, 64))
    acc = acc + xb * wsl[0:1, :] + yb * wsl[1:2, :] + zb * wsl[2:3, :]
    acc = acc + (xb * yb) * wsq[0:1, :] + (yb * zb) * wsq[1:2, :] \
        + (zb * zb) * wsq[2:3, :] + (xb * zb) * wsq[3:4, :] \
        + (xb * xb) * wsq[4:5, :] + (yb * yb) * wsq[5:6, :]
    acc = acc + b2tab[0:1, :] + bondf[...] * (b2tab[1:2, :] - b2tab[0:1, :])
    e_out[...] = _silu(acc + bias[...])
    fc = 0.5 * (jnp.cos((jnp.pi / R_MAX) * jnp.minimum(r, R_MAX)) + 1.0)
    fc = fc * (r < R_MAX).astype(jnp.float32)
    gid = (pl.program_id(0) * EBK
           + lax.broadcasted_iota(jnp.int32, (EBK, 1), 0))
    fc_out[...] = fc * (gid < N_EDGES).astype(jnp.float32)


def _edge_step_body(xab, e, fc, w1c, b1, w2m, b2c, w2g,
                    e_out, ma_out, mb_out):
    pre = (xab[:, 0:64] + xab[:, 64:128]
           + _dot(e[...], w1c[...], preferred_element_type=jnp.float32)
           + b1[...])
    m = _silu(pre)
    e_out[...] = e[...] + m
    m2 = _dot(m, w2m[...], preferred_element_type=jnp.float32) \
        + b2c[:, 0:64]
    g = _dot(m, w2g[...], preferred_element_type=jnp.float32) \
        + b2c[:, 64:65]
    msg = m2 * jax.nn.sigmoid(g) * fc[...]
    ma_out[...] = msg[:, 0:32]
    mb_out[...] = msg[:, 32:64]


def _node_step_body(h, agga, aggb, wn1h, wn1a, wn1b, bn1, wn2, bn2,
                    w1a, w1b, h_out, a_out, b_out):
    t = (_dot(h[...], wn1h[...], preferred_element_type=jnp.float32)
         + _dot(agga[...], wn1a[...], preferred_element_type=jnp.float32)
         + _dot(aggb[...], wn1b[...], preferred_element_type=jnp.float32)
         + bn1[...])
    hn = h[...] + _dot(_silu(t), wn2[...],
                          preferred_element_type=jnp.float32) + bn2[...]
    h_out[...] = hn
    a_out[...] = _dot(hn, w1a[...], preferred_element_type=jnp.float32)
    b_out[...] = _dot(hn, w1b[...], preferred_element_type=jnp.float32)


def _node_last_body(h, agga, aggb, wn1h, wn1a, wn1b, bn1, wn2, bn2,
                    wh1, bh1, wh2, bh2, out):
    t = (_dot(h[...], wn1h[...], preferred_element_type=jnp.float32)
         + _dot(agga[...], wn1a[...], preferred_element_type=jnp.float32)
         + _dot(aggb[...], wn1b[...], preferred_element_type=jnp.float32)
         + bn1[...])
    hn = h[...] + _dot(_silu(t), wn2[...],
                          preferred_element_type=jnp.float32) + bn2[...]
    y = _silu(_dot(hn, wh1[...], preferred_element_type=jnp.float32)
              + bh1[...])
    out[...] = _dot(y, wh2[...], preferred_element_type=jnp.float32) \
        + bh2[...]


def _row_spec(rows, cols):
    return pl.BlockSpec((rows, cols), lambda i: (i, 0))


def _rep_spec(shape):
    return pl.BlockSpec(shape, lambda i: tuple(0 for _ in shape))


def kernel(pos, atom_type_index, atom_code_index, residue_code_index,
           residue_sequence_index, bond_mask, senders, receivers, batch,
           num_graphs, c_noise, c_in, params):
    del batch, num_graphs, c_noise
    f32 = jnp.float32

    up64 = jnp.zeros((N_PAD, 64), f32)
    up64 = up64.at[:N_NODES, :3].set(pos / c_in[0])

    codes = jnp.stack([
        atom_type_index,
        20 + atom_code_index,
        30 + residue_code_index,
        jnp.full((N_NODES,), 55, jnp.int32),
    ], axis=1)
    codes = jnp.concatenate(
        [codes, jnp.zeros((N_PAD - N_NODES, 4), jnp.int32)], axis=0)

    epad = E_PAD - N_EDGES
    s2d = jnp.concatenate(
        [senders, jnp.zeros((epad,), jnp.int32)]).reshape(N_CHUNKS, CHUNK)
    r2d = jnp.concatenate(
        [receivers, jnp.zeros((epad,), jnp.int32)]).reshape(N_CHUNKS, CHUNK)
    bondf = jnp.concatenate(
        [bond_mask.astype(f32), jnp.zeros((epad,), f32)]).reshape(E_PAD, 1)

    p = params
    w_node = p['W_node_enc']
    ctab = jnp.zeros((64, 64), f32)
    ctab = ctab.at[0:20].set(p['emb_atom_type'] @ w_node[0:32])
    ctab = ctab.at[20:30].set(p['emb_atom_code'] @ w_node[32:48])
    ctab = ctab.at[30:55].set(p['emb_res_code'] @ w_node[48:64])
    ctab = ctab.at[55:56].set(p['emb_res_idx'] @ w_node[64:80])
    b_node = p['b_node_enc'].reshape(1, 64)

    w_edge = p['W_edge_enc']
    wr = w_edge[0:8]
    ws = w_edge[8:17]
    s3, s15, s5 = jnp.sqrt(3.0), jnp.sqrt(15.0), jnp.sqrt(5.0)
    wsl = jnp.stack([s3 * ws[1], s3 * ws[2], s3 * ws[3]])
    wsq = jnp.stack([s15 * ws[4], s15 * ws[5], 1.5 * s5 * ws[6],
                     s15 * ws[7], 0.5 * s15 * ws[8], -0.5 * s15 * ws[8]])
    b2tab = p['emb_bond'] @ w_edge[17:33]
    b_edge = (p['b_edge_enc'] + ws[0] - 0.5 * s5 * ws[6]).reshape(1, 64)
    sel = jnp.zeros((16, 192), f32)
    sel = sel.at[0, 0:64].set(1.0).at[1, 64:128].set(1.0)
    sel = sel.at[2, 128:192].set(1.0)

    steps_w = []
    for s in range(STEPS):
        ps_ = p['steps'][s]
        steps_w.append(dict(
            w1a=ps_['W_e1'][0:64], w1b=ps_['W_e1'][64:128],
            w1c=ps_['W_e1'][128:192], b1=ps_['b_e1'].reshape(1, 64),
            w2m=ps_['W_e2'][:, 0:64], w2g=ps_['W_e2'][:, 64:65],
            b2c=jnp.zeros((1, 128), f32).at[0, :65].set(ps_['b_e2']),
            wn1h=ps_['W_n1'][0:64], wn1a=ps_['W_n1'][64:96],
            wn1b=ps_['W_n1'][96:128], bn1=ps_['b_n1'].reshape(1, 64),
            wn2=ps_['W_n2'], bn2=ps_['b_n2'].reshape(1, 64),
        ))
    wh2 = jnp.zeros((64, 8), f32).at[:, :3].set(p['W_h2'])
    bh2 = jnp.zeros((1, 8), f32).at[0, :3].set(p['b_h2'])
    bh1 = p['b_h1'].reshape(1, 64)

    zinit = jnp.zeros((NROWS_T, 32), f32)

    ngrid = N_PAD // NBK
    egrid = E_PAD // EBK
    nfull = _row_spec(NBK, 64)
    nhalf = _row_spec(NBK, 32)
    efull = _row_spec(EBK, 64)
    ehalf = _row_spec(EBK, 32)
    e1col = _row_spec(EBK, 1)
    w64 = _rep_spec((64, 64))
    b64 = _rep_spec((1, 64))

    h, a_mat, b_mat = pl.pallas_call(
        _node_encode_body,
        grid=(ngrid,),
        in_specs=[_row_spec(NBK, 4), w64, b64, w64, w64],
        out_specs=[nfull, nfull, nfull],
        out_shape=[jax.ShapeDtypeStruct((N_PAD, 64), f32)] * 3,
    )(codes, ctab, b_node, steps_w[0]['w1a'], steps_w[0]['w1b'])

    pp = _gather2_64(up64, up64, s2d, r2d)
    e, fc = pl.pallas_call(
        _edge_encode_body,
        grid=(egrid,),
        in_specs=[_row_spec(EBK, 128), e1col,
                  _rep_spec((8, 64)), _rep_spec((3, 64)), _rep_spec((6, 64)),
                  _rep_spec((2, 64)), b64, _rep_spec((16, 192))],
        out_specs=[efull, e1col],
        out_shape=[jax.ShapeDtypeStruct((E_PAD, 64), f32),
                   jax.ShapeDtypeStruct((E_PAD, 1), f32)],
    )(pp, bondf, wr, wsl, wsq, b2tab, b_edge, sel)

    out = None
    for s in range(STEPS):
        sw = steps_w[s]
        xab = _gather2_64(a_mat, b_mat, s2d, r2d)
        e, msga, msgb = pl.pallas_call(
            _edge_step_body,
            grid=(egrid,),
            in_specs=[_row_spec(EBK, 128), efull, e1col, w64, b64, w64,
                      _rep_spec((1, 128)), _rep_spec((64, 1))],
            out_specs=[efull, ehalf, ehalf],
            out_shape=[jax.ShapeDtypeStruct((E_PAD, 64), f32),
                       jax.ShapeDtypeStruct((E_PAD, 32), f32),
                       jax.ShapeDtypeStruct((E_PAD, 32), f32)],
            input_output_aliases={1: 0},
        )(xab, e, fc, sw['w1c'], sw['b1'], sw['w2m'], sw['b2c'], sw['w2g'])

        agga, aggb = _scatter_agg(msga, msgb, r2d, zinit)

        if s < STEPS - 1:
            nw = steps_w[s + 1]
            h, a_mat, b_mat = pl.pallas_call(
                _node_step_body,
                grid=(ngrid,),
                in_specs=[nfull, nhalf, nhalf, w64, _rep_spec((32, 64)),
                          _rep_spec((32, 64)), b64, w64, b64, w64, w64],
                out_specs=[nfull, nfull, nfull],
                out_shape=[jax.ShapeDtypeStruct((N_PAD, 64), f32)] * 3,
            )(h, agga, aggb, sw['wn1h'], sw['wn1a'], sw['wn1b'], sw['bn1'],
              sw['wn2'], sw['bn2'], nw['w1a'], nw['w1b'])
        else:
            out = pl.pallas_call(
                _node_last_body,
                grid=(ngrid,),
                in_specs=[nfull, nhalf, nhalf, w64, _rep_spec((32, 64)),
                          _rep_spec((32, 64)), b64, w64, b64, w64, b64,
                          _rep_spec((64, 8)), _rep_spec((1, 8))],
                out_specs=[_row_spec(NBK, 8)],
                out_shape=[jax.ShapeDtypeStruct((N_PAD, 8), f32)],
            )(h, agga, aggb, sw['wn1h'], sw['wn1a'], sw['wn1b'], sw['bn1'],
              sw['wn2'], sw['bn2'], p['W_h1'], bh1, wh2, bh2)[0]

    return out[:N_NODES, :3]


# speedup vs baseline: 1.3715x; 1.2137x over previous
import functools

import jax
import jax.numpy as jnp
from jax import lax
from jax.experimental import pallas as pl
from jax.experimental.pallas import tpu as pltpu
from jax.experimental.pallas import tpu_sc as plsc

N_NODES = 50000
N_EDGES = 800000
LATENT = 64
STEPS = 3
R_MAX = 5.0
N_BASES = 8

NBK = 256
EBK = 1024
N_PAD = 50176
E_PAD = 819200
CHUNK = 128
N_CHUNKS = E_PAD // CHUNK
NC, NS = 2, 16
NW = NC * NS
CPW = N_CHUNKS // NW
CPT = N_CHUNKS // NS
NROWS_T = N_PAD // NS
IBLK = 40

def _get_mesh():
    return plsc.VectorSubcoreMesh(
        core_axis_name="c", subcore_axis_name="s",
        num_cores=NC, num_subcores=NS)


def _silu(x):
    return x * jax.nn.sigmoid(x)


def _dot(a, b, preferred_element_type=jnp.float32):
    return jnp.dot(a, b, preferred_element_type=preferred_element_type)


@functools.cache
def _make_gather2(d):
    @functools.partial(
        pl.kernel,
        out_type=jax.ShapeDtypeStruct((E_PAD, 2 * d), jnp.float32),
        mesh=_get_mesh(),
        scratch_types=[
            pltpu.VMEM((CPW, CHUNK), jnp.int32),
            pltpu.VMEM((CPW, CHUNK), jnp.int32),
            pltpu.VMEM((2, CHUNK, d), jnp.float32),
            pltpu.VMEM((2, CHUNK, d), jnp.float32),
            [pltpu.SemaphoreType.DMA] * 2,
            [pltpu.SemaphoreType.DMA] * 2,
        ],
        compiler_params=pltpu.CompilerParams(use_tc_tiling_on_sc=False),
    )
    def gather2(ta, tb, sidx, ridx, outab, idx_s, idx_r, bufa, bufb,
                gsem, wsem):
        wid = lax.axis_index("s") * NC + lax.axis_index("c")
        cbase = wid * CPW
        pltpu.sync_copy(sidx.at[pl.ds(cbase, CPW)], idx_s)
        pltpu.sync_copy(ridx.at[pl.ds(cbase, CPW)], idx_r)

        def fire(j, slot):
            pltpu.async_copy(ta.at[idx_s.at[j]], bufa.at[slot], gsem[slot])
            pltpu.async_copy(tb.at[idx_r.at[j]], bufb.at[slot], gsem[slot])

        def out_refs(j):
            ebase = (cbase + j) * CHUNK
            return (outab.at[pl.ds(ebase, CHUNK), pl.ds(0, d)],
                    outab.at[pl.ds(ebase, CHUNK), pl.ds(d, d)])

        def wait_gathers(j, slot):
            pltpu.make_async_copy(
                ta.at[idx_s.at[j]], bufa.at[slot], gsem[slot]).wait()
            pltpu.make_async_copy(
                tb.at[idx_r.at[j]], bufb.at[slot], gsem[slot]).wait()

        def drain_writes(j, slot):
            oa, ob = out_refs(j)
            pltpu.make_async_copy(bufa.at[slot], oa, wsem[slot]).wait()
            pltpu.make_async_copy(bufb.at[slot], ob, wsem[slot]).wait()

        def issue_writes(j, slot):
            oa, ob = out_refs(j)
            pltpu.async_copy(bufa.at[slot], oa, wsem[slot])
            pltpu.async_copy(bufb.at[slot], ob, wsem[slot])

        fire(0, 0)

        @pl.loop(0, CPW // 2)
        def _(jj):
            j0 = jj * 2
            wait_gathers(j0, 0)

            @pl.when(jj > 0)
            def _():
                drain_writes(j0 - 1, 1)

            fire(j0 + 1, 1)
            issue_writes(j0, 0)

            wait_gathers(j0 + 1, 1)
            drain_writes(j0, 0)

            @pl.when(jj < CPW // 2 - 1)
            def _():
                fire(j0 + 2, 0)

            issue_writes(j0 + 1, 1)

        drain_writes(CPW - 1, 1)

    return gather2


def _gather2_64(ta, tb, sidx, ridx):
    return _make_gather2(64)(ta, tb, sidx, ridx)


@functools.cache
def _make_scatter():
    @functools.partial(
        pl.kernel,
        out_type=(jax.ShapeDtypeStruct((N_PAD, 32), jnp.float32),
                  jax.ShapeDtypeStruct((N_PAD, 32), jnp.float32)),
        mesh=_get_mesh(),
        scratch_types=[
            pltpu.VMEM_SHARED((N_PAD, 32), jnp.float32),
            pltpu.VMEM((IBLK, CHUNK), jnp.int32),
            pltpu.VMEM((2, CHUNK, 32), jnp.float32),
            [pltpu.SemaphoreType.DMA] * 2,
        ],
        compiler_params=pltpu.CompilerParams(use_tc_tiling_on_sc=False),
    )
    def scatter_agg(msga, msgb, ridx, zinit, agga, aggb, spmem, idx_v, buf,
                    sem):
        c = lax.axis_index("c")
        s = lax.axis_index("s")
        rbase = s * NROWS_T

        pltpu.sync_copy(zinit, spmem.at[pl.ds(rbase, NROWS_T)])
        plsc.subcore_barrier()

        def accum(msg):
            def fire(cb, j, slot):
                pltpu.async_copy(
                    msg.at[pl.ds((cb + j) * CHUNK, CHUNK)],
                    buf.at[slot], sem[slot])

            def wait(cb, j, slot):
                pltpu.make_async_copy(
                    msg.at[pl.ds((cb + j) * CHUNK, CHUNK)],
                    buf.at[slot], sem[slot]).wait()

            @pl.loop(0, CPT // IBLK)
            def _(b):
                cb = s * CPT + b * IBLK
                pltpu.sync_copy(ridx.at[pl.ds(cb, IBLK)], idx_v)
                fire(cb, 0, 0)

                @pl.loop(0, IBLK // 2)
                def _(jj):
                    j0 = jj * 2
                    wait(cb, j0, 0)
                    fire(cb, j0 + 1, 1)
                    pltpu.sync_copy(
                        buf.at[0], spmem.at[idx_v.at[j0]], add=True)
                    wait(cb, j0 + 1, 1)

                    @pl.when(jj < IBLK // 2 - 1)
                    def _():
                        fire(cb, j0 + 2, 0)

                    pltpu.sync_copy(
                        buf.at[1], spmem.at[idx_v.at[j0 + 1]], add=True)

        @pl.when(c == 0)
        def _():
            accum(msga)

        @pl.when(c == 1)
        def _():
            accum(msgb)

        plsc.subcore_barrier()

        @pl.when(c == 0)
        def _():
            pltpu.sync_copy(spmem.at[pl.ds(rbase, NROWS_T)],
                            agga.at[pl.ds(rbase, NROWS_T)])

        @pl.when(c == 1)
        def _():
            pltpu.sync_copy(spmem.at[pl.ds(rbase, NROWS_T)],
                            aggb.at[pl.ds(rbase, NROWS_T)])

    return scatter_agg


def _scatter_agg(msga, msgb, ridx, zinit):
    return _make_scatter()(msga, msgb, ridx, zinit)


def _node_encode_body(codes, ctab, bias, w1a, w1b, h_out, a_out, b_out):
    iot = lax.broadcasted_iota(jnp.int32, (NBK, 64), 1)
    m = jnp.zeros((NBK, 64), jnp.float32)
    for k in range(4):
        m = m + (codes[:, k:k + 1] == iot).astype(jnp.float32)
    h = _silu(_dot(m, ctab[...], preferred_element_type=jnp.float32)
              + bias[...])
    h_out[...] = h
    a_out[...] = _dot(h, w1a[...], preferred_element_type=jnp.float32)
    b_out[...] = _dot(h, w1b[...], preferred_element_type=jnp.float32)


def _edge_encode_body(pp, bondf, wr, wsl, wsq, b2tab, bias, sel,
                      e_out, fc_out):
    del sel
    d = pp[:, 0:16] - pp[:, 16:32]
    dt = jnp.transpose(d[:, 0:8], (1, 0))
    r2t = jnp.sum(dt * dt, axis=0, keepdims=True)
    rt = jnp.sqrt(r2t)
    rst = jnp.maximum(rt, 1e-6)
    rinvt = 1.0 / rst
    n = (lax.broadcasted_iota(jnp.int32, (N_BASES, EBK), 0) + 1
         ).astype(jnp.float32)
    rbft = (jnp.sqrt(2.0 / R_MAX) * jnp.sin(n * (jnp.pi / R_MAX) * rst)
            * rinvt)
    acc = lax.dot_general(rbft, wr[...], (((0,), (0,)), ((), ())),
                          preferred_element_type=jnp.float32)
    fct = 0.5 * (jnp.cos((jnp.pi / R_MAX) * jnp.minimum(rt, R_MAX)) + 1.0)
    fct = fct * (rt < R_MAX).astype(jnp.float32)
    gid = (pl.program_id(0) * EBK
           + lax.broadcasted_iota(jnp.int32, (1, EBK), 1))
    fct = fct * (gid < N_EDGES).astype(jnp.float32)
    fc_out[...] = jnp.transpose(fct, (1, 0))
    rinv = jnp.transpose(rinvt, (1, 0))
    rinvb = jnp.broadcast_to(rinv, (EBK, 64))
    xb = jnp.broadcast_to(d[:, 0:1], (EBK, 64)) * rinvb
    yb = jnp.broadcast_to(d[:, 1:2], (EBK, 64)) * rinvb
    zb = jnp.broadcast_to(d[:, 2:3], (EBK, 64)) * rinvb
    acc = acc + xb * wsl[0:1, :] + yb * wsl[1:2, :] + zb * wsl[2:3, :]
    acc = acc + (xb * yb) * wsq[0:1, :] + (yb * zb) * wsq[1:2, :] \
        + (zb * zb) * wsq[2:3, :] + (xb * zb) * wsq[3:4, :] \
        + (xb * xb) * wsq[4:5, :] + (yb * yb) * wsq[5:6, :]
    acc = acc + b2tab[0:1, :] + bondf[...] * (b2tab[1:2, :] - b2tab[0:1, :])
    e_out[...] = _silu(acc + bias[...])


def _edge_step_body(xab, e, fc, w1c, b1, w2m, b2c, w2g,
                    e_out, ma_out, mb_out):
    pre = (xab[:, 0:64] + xab[:, 64:128]
           + _dot(e[...], w1c[...], preferred_element_type=jnp.float32)
           + b1[...])
    m = _silu(pre)
    e_out[...] = e[...] + m
    m2 = _dot(m, w2m[...], preferred_element_type=jnp.float32) \
        + b2c[:, 0:64]
    g = _dot(m, w2g[...], preferred_element_type=jnp.float32) \
        + b2c[:, 64:65]
    msg = m2 * jax.nn.sigmoid(g) * fc[...]
    ma_out[...] = msg[:, 0:32]
    mb_out[...] = msg[:, 32:64]


def _node_step_body(h, agga, aggb, wn1h, wn1a, wn1b, bn1, wn2, bn2,
                    w1a, w1b, h_out, a_out, b_out):
    t = (_dot(h[...], wn1h[...], preferred_element_type=jnp.float32)
         + _dot(agga[...], wn1a[...], preferred_element_type=jnp.float32)
         + _dot(aggb[...], wn1b[...], preferred_element_type=jnp.float32)
         + bn1[...])
    hn = h[...] + _dot(_silu(t), wn2[...],
                          preferred_element_type=jnp.float32) + bn2[...]
    h_out[...] = hn
    a_out[...] = _dot(hn, w1a[...], preferred_element_type=jnp.float32)
    b_out[...] = _dot(hn, w1b[...], preferred_element_type=jnp.float32)


def _node_last_body(h, agga, aggb, wn1h, wn1a, wn1b, bn1, wn2, bn2,
                    wh1, bh1, wh2, bh2, out):
    t = (_dot(h[...], wn1h[...], preferred_element_type=jnp.float32)
         + _dot(agga[...], wn1a[...], preferred_element_type=jnp.float32)
         + _dot(aggb[...], wn1b[...], preferred_element_type=jnp.float32)
         + bn1[...])
    hn = h[...] + _dot(_silu(t), wn2[...],
                          preferred_element_type=jnp.float32) + bn2[...]
    y = _silu(_dot(hn, wh1[...], preferred_element_type=jnp.float32)
              + bh1[...])
    out[...] = _dot(y, wh2[...], preferred_element_type=jnp.float32) \
        + bh2[...]


def _row_spec(rows, cols):
    return pl.BlockSpec((rows, cols), lambda i: (i, 0))


def _rep_spec(shape):
    return pl.BlockSpec(shape, lambda i: tuple(0 for _ in shape))


def kernel(pos, atom_type_index, atom_code_index, residue_code_index,
           residue_sequence_index, bond_mask, senders, receivers, batch,
           num_graphs, c_noise, c_in, params):
    del batch, num_graphs, c_noise
    f32 = jnp.float32

    up16 = jnp.zeros((N_PAD, 16), f32)
    up16 = up16.at[:N_NODES, :3].set(pos / c_in[0])

    codes = jnp.stack([
        atom_type_index,
        20 + atom_code_index,
        30 + residue_code_index,
        jnp.full((N_NODES,), 55, jnp.int32),
    ], axis=1)
    codes = jnp.concatenate(
        [codes, jnp.zeros((N_PAD - N_NODES, 4), jnp.int32)], axis=0)

    epad = E_PAD - N_EDGES
    s2d = jnp.concatenate(
        [senders, jnp.zeros((epad,), jnp.int32)]).reshape(N_CHUNKS, CHUNK)
    r2d = jnp.concatenate(
        [receivers, jnp.zeros((epad,), jnp.int32)]).reshape(N_CHUNKS, CHUNK)
    bondf = jnp.concatenate(
        [bond_mask.astype(f32), jnp.zeros((epad,), f32)]).reshape(E_PAD, 1)

    p = params
    w_node = p['W_node_enc']
    ctab = jnp.zeros((64, 64), f32)
    ctab = ctab.at[0:20].set(p['emb_atom_type'] @ w_node[0:32])
    ctab = ctab.at[20:30].set(p['emb_atom_code'] @ w_node[32:48])
    ctab = ctab.at[30:55].set(p['emb_res_code'] @ w_node[48:64])
    ctab = ctab.at[55:56].set(p['emb_res_idx'] @ w_node[64:80])
    b_node = p['b_node_enc'].reshape(1, 64)

    w_edge = p['W_edge_enc']
    wr = w_edge[0:8]
    ws = w_edge[8:17]
    s3, s15, s5 = jnp.sqrt(3.0), jnp.sqrt(15.0), jnp.sqrt(5.0)
    wsl = jnp.stack([s3 * ws[1], s3 * ws[2], s3 * ws[3]])
    wsq = jnp.stack([s15 * ws[4], s15 * ws[5], 1.5 * s5 * ws[6],
                     s15 * ws[7], 0.5 * s15 * ws[8], -0.5 * s15 * ws[8]])
    b2tab = p['emb_bond'] @ w_edge[17:33]
    b_edge = (p['b_edge_enc'] + ws[0] - 0.5 * s5 * ws[6]).reshape(1, 64)
    sel = jnp.zeros((16, 192), f32)
    sel = sel.at[0, 0:64].set(1.0).at[1, 64:128].set(1.0)
    sel = sel.at[2, 128:192].set(1.0)

    steps_w = []
    for s in range(STEPS):
        ps_ = p['steps'][s]
        steps_w.append(dict(
            w1a=ps_['W_e1'][0:64], w1b=ps_['W_e1'][64:128],
            w1c=ps_['W_e1'][128:192], b1=ps_['b_e1'].reshape(1, 64),
            w2m=ps_['W_e2'][:, 0:64], w2g=ps_['W_e2'][:, 64:65],
            b2c=jnp.zeros((1, 128), f32).at[0, :65].set(ps_['b_e2']),
            wn1h=ps_['W_n1'][0:64], wn1a=ps_['W_n1'][64:96],
            wn1b=ps_['W_n1'][96:128], bn1=ps_['b_n1'].reshape(1, 64),
            wn2=ps_['W_n2'], bn2=ps_['b_n2'].reshape(1, 64),
        ))
    wh2 = jnp.zeros((64, 8), f32).at[:, :3].set(p['W_h2'])
    bh2 = jnp.zeros((1, 8), f32).at[0, :3].set(p['b_h2'])
    bh1 = p['b_h1'].reshape(1, 64)

    zinit = jnp.zeros((NROWS_T, 32), f32)

    ngrid = N_PAD // NBK
    egrid = E_PAD // EBK
    nfull = _row_spec(NBK, 64)
    nhalf = _row_spec(NBK, 32)
    efull = _row_spec(EBK, 64)
    ehalf = _row_spec(EBK, 32)
    e1col = _row_spec(EBK, 1)
    w64 = _rep_spec((64, 64))
    b64 = _rep_spec((1, 64))

    h, a_mat, b_mat = pl.pallas_call(
        _node_encode_body,
        grid=(ngrid,),
        in_specs=[_row_spec(NBK, 4), w64, b64, w64, w64],
        out_specs=[nfull, nfull, nfull],
        out_shape=[jax.ShapeDtypeStruct((N_PAD, 64), f32)] * 3,
    )(codes, ctab, b_node, steps_w[0]['w1a'], steps_w[0]['w1b'])

    pp = _make_gather2(16)(up16, up16, s2d, r2d)
    e, fc = pl.pallas_call(
        _edge_encode_body,
        grid=(egrid,),
        in_specs=[_row_spec(EBK, 32), e1col,
                  _rep_spec((8, 64)), _rep_spec((3, 64)), _rep_spec((6, 64)),
                  _rep_spec((2, 64)), b64, _rep_spec((16, 192))],
        out_specs=[efull, e1col],
        out_shape=[jax.ShapeDtypeStruct((E_PAD, 64), f32),
                   jax.ShapeDtypeStruct((E_PAD, 1), f32)],
    )(pp, bondf, wr, wsl, wsq, b2tab, b_edge, sel)

    out = None
    for s in range(STEPS):
        sw = steps_w[s]
        xab = _gather2_64(a_mat, b_mat, s2d, r2d)
        e, msga, msgb = pl.pallas_call(
            _edge_step_body,
            grid=(egrid,),
            in_specs=[_row_spec(EBK, 128), efull, e1col, w64, b64, w64,
                      _rep_spec((1, 128)), _rep_spec((64, 1))],
            out_specs=[efull, ehalf, ehalf],
            out_shape=[jax.ShapeDtypeStruct((E_PAD, 64), f32),
                       jax.ShapeDtypeStruct((E_PAD, 32), f32),
                       jax.ShapeDtypeStruct((E_PAD, 32), f32)],
            input_output_aliases={1: 0},
        )(xab, e, fc, sw['w1c'], sw['b1'], sw['w2m'], sw['b2c'], sw['w2g'])

        agga, aggb = _scatter_agg(msga, msgb, r2d, zinit)

        if s < STEPS - 1:
            nw = steps_w[s + 1]
            h, a_mat, b_mat = pl.pallas_call(
                _node_step_body,
                grid=(ngrid,),
                in_specs=[nfull, nhalf, nhalf, w64, _rep_spec((32, 64)),
                          _rep_spec((32, 64)), b64, w64, b64, w64, w64],
                out_specs=[nfull, nfull, nfull],
                out_shape=[jax.ShapeDtypeStruct((N_PAD, 64), f32)] * 3,
            )(h, agga, aggb, sw['wn1h'], sw['wn1a'], sw['wn1b'], sw['bn1'],
              sw['wn2'], sw['bn2'], nw['w1a'], nw['w1b'])
        else:
            out = pl.pallas_call(
                _node_last_body,
                grid=(ngrid,),
                in_specs=[nfull, nhalf, nhalf, w64, _rep_spec((32, 64)),
                          _rep_spec((32, 64)), b64, w64, b64, w64, b64,
                          _rep_spec((64, 8)), _rep_spec((1, 8))],
                out_specs=[_row_spec(NBK, 8)],
                out_shape=[jax.ShapeDtypeStruct((N_PAD, 8), f32)],
            )(h, agga, aggb, sw['wn1h'], sw['wn1a'], sw['wn1b'], sw['bn1'],
              sw['wn2'], sw['bn2'], p['W_h1'], bh1, wh2, bh2)[0]

    return out[:N_NODES, :3]
